# baseline (speedup 1.0000x reference)
import functools

import jax
import jax.numpy as jnp
from jax import lax
from jax.experimental import pallas as pl
from jax.experimental.pallas import tpu as pltpu
from jax.experimental.pallas import tpu_sc as plsc

N = 10000
E = 320000
D = 128
C = 64

NW = 32
NPW = 320
NPAD = NW * NPW
EB = 6400
NCHUNK = E // EB
G = 128


def _segmax_body(h_hbm, src_hbm, dst_hbm, out_hbm,
                 src_v, dst_v, cs_src, cs_ldst, gbuf, agg, sem):
    cid = lax.axis_index("c")
    sid = lax.axis_index("s")
    wid = sid * 2 + cid
    lo = wid * NPW

    zf = jnp.zeros((16,), jnp.float32)
    zi = jnp.zeros((16,), jnp.int32)

    def zrow(r, carry):
        for cc in range(D // 16):
            agg[r, pl.ds(cc * 16, 16)] = zf
        return carry
    lax.fori_loop(0, NPW + 1, zrow, 0)

    def zsrc(i, carry):
        cs_src[pl.ds(i * 16, 16)] = zi
        return carry
    lax.fori_loop(0, (EB + 16) // 16, zsrc, 0)

    def chunk_body(ch, carry):
        base = ch * EB
        pltpu.sync_copy(src_hbm.at[pl.ds(base, EB)], src_v)
        pltpu.sync_copy(dst_hbm.at[pl.ds(base, EB)], dst_v)

        def compact(i, cnt):
            d = dst_v[pl.ds(i * 16, 16)]
            sv = src_v[pl.ds(i * 16, 16)]
            m = (d >= lo) & (d < lo + NPW)
            csum = plsc.cumsum(jnp.where(m, 1, 0))
            idx = cnt + csum - 1
            plsc.store_scatter(cs_src, [idx], sv, mask=m)
            plsc.store_scatter(cs_ldst, [idx], d - lo, mask=m)
            return cnt + csum[15]
        cnt = lax.fori_loop(0, EB // 16, compact, jnp.int32(0))

        def group(g, carry):
            gbase = g * G
            pltpu.async_copy(h_hbm.at[cs_src.at[pl.ds(gbase, G)]], gbuf, sem).wait()
            mrows = jnp.minimum(cnt - gbase, G)

            def upd(j, carry2):
                ldj = cs_ldst[pl.ds(gbase + j, 16)][0]
                for cc in range(D // 16):
                    sl = pl.ds(cc * 16, 16)
                    agg[ldj, sl] = jnp.maximum(agg[ldj, sl], gbuf[j, sl])
                return carry2
            lax.fori_loop(0, mrows, upd, 0)
            return carry
        lax.fori_loop(0, (cnt + G - 1) // G, group, 0)
        return carry
    lax.fori_loop(0, NCHUNK, chunk_body, 0)

    pltpu.sync_copy(agg.at[pl.ds(0, NPW)], out_hbm.at[pl.ds(lo, NPW)])


_segmax = pl.kernel(
    _segmax_body,
    mesh=plsc.VectorSubcoreMesh(core_axis_name="c", subcore_axis_name="s"),
    compiler_params=pltpu.CompilerParams(needs_layout_passes=False),
    out_type=jax.ShapeDtypeStruct((NPAD, D), jnp.float32),
    scratch_types=[
        pltpu.VMEM((EB,), jnp.int32),
        pltpu.VMEM((EB,), jnp.int32),
        pltpu.VMEM((EB + 16,), jnp.int32),
        pltpu.VMEM((EB + 16,), jnp.int32),
        pltpu.VMEM((G, D), jnp.float32),
        pltpu.VMEM((NPW + 1, D), jnp.float32),
        pltpu.SemaphoreType.DMA,
    ],
)


_DOT = functools.partial(
    lax.dot_general,
    dimension_numbers=(((1,), (1,)), ((), ())),
    preferred_element_type=jnp.float32,
)


def _lrelu(t):
    return jnp.where(t > 0, t, 0.01 * t)


def _mm_pool_body(x_ref, w_ref, b_ref, h_ref):
    h_ref[...] = jnp.maximum(_DOT(x_ref[...], w_ref[...]) + b_ref[...], 0.0)


def _mm_mid_body(x_ref, a_ref, ws_ref, wn_ref, b_ref, wp2_ref, bp2_ref,
                 y_ref, h2_ref):
    y = _lrelu(_DOT(x_ref[...], ws_ref[...]) + _DOT(a_ref[...], wn_ref[...])
               + b_ref[...])
    y_ref[...] = y
    h2_ref[...] = jnp.maximum(_DOT(y, wp2_ref[...]) + bp2_ref[...], 0.0)


def _mm_head_body(y1_ref, a2_ref, ws2_ref, wn2_ref, b2_ref,
                  l1w_ref, l1b_ref, l2w_ref, l2b_ref, l3w_ref, l3b_ref,
                  l4w_ref, l4b_ref, l5w_ref, l5b_ref, out_ref):
    h = _lrelu(_DOT(y1_ref[...], ws2_ref[...]) + _DOT(a2_ref[...], wn2_ref[...])
               + b2_ref[...])
    h = _lrelu(_DOT(h, l1w_ref[...]) + l1b_ref[...])
    h = _lrelu(_DOT(h, l2w_ref[...]) + l2b_ref[...])
    h = _lrelu(_DOT(h, l3w_ref[...]) + l3b_ref[...])
    h = _lrelu(_DOT(h, l4w_ref[...]) + l4b_ref[...])
    out_ref[...] = _DOT(h, l5w_ref[...]) + l5b_ref[...]


_RB = 2000


def _row_spec(width):
    return pl.BlockSpec((_RB, width), lambda i: (i, 0))


def _full_spec(shape):
    return pl.BlockSpec(shape, lambda i: (0,) * len(shape))


def _w():
    return _full_spec((D, D))


def _b():
    return _full_spec((1, D))


_mm_pool = pl.pallas_call(
    _mm_pool_body,
    grid=(N // _RB,),
    in_specs=[_row_spec(D), _w(), _b()],
    out_specs=_row_spec(D),
    out_shape=jax.ShapeDtypeStruct((N, D), jnp.float32),
)

_mm_mid = pl.pallas_call(
    _mm_mid_body,
    grid=(N // _RB,),
    in_specs=[_row_spec(D), _row_spec(D), _w(), _w(), _b(), _w(), _b()],
    out_specs=[_row_spec(D), _row_spec(D)],
    out_shape=[jax.ShapeDtypeStruct((N, D), jnp.float32),
               jax.ShapeDtypeStruct((N, D), jnp.float32)],
)

_mm_head = pl.pallas_call(
    _mm_head_body,
    grid=(N // _RB,),
    in_specs=[_row_spec(D), _row_spec(D), _w(), _w(), _b(),
              _w(), _b(), _w(), _b(), _w(), _b(), _w(), _b(),
              _full_spec((C, D)), _full_spec((1, C))],
    out_specs=_row_spec(C),
    out_shape=jax.ShapeDtypeStruct((N, C), jnp.float32),
)


def kernel(x, edge_index, Wp1, bp1, Wn1, Ws1, b1, Wp2, bp2, Wn2, Ws2, b2,
           L1W, L1b, L2W, L2b, L3W, L3b, L4W, L4b, L5W, L5b):
    src = edge_index[0]
    dst = edge_index[1]

    h1 = _mm_pool(x, Wp1, bp1.reshape(1, D))
    agg1 = _segmax(h1, src, dst)[:N]
    y1, h2 = _mm_mid(x, agg1, Ws1, Wn1, b1.reshape(1, D), Wp2,
                     bp2.reshape(1, D))
    agg2 = _segmax(h2, src, dst)[:N]
    return _mm_head(y1, agg2, Ws2, Wn2, b2.reshape(1, D),
                    L1W, L1b.reshape(1, D), L2W, L2b.reshape(1, D),
                    L3W, L3b.reshape(1, D), L4W, L4b.reshape(1, D),
                    L5W, L5b.reshape(1, C))
